# baseline (speedup 1.0000x reference)
import functools
import jax
import jax.numpy as jnp
from jax import lax
from jax.experimental import pallas as pl
from jax.experimental.pallas import tpu as pltpu

H = 128


def _msg_body(g_ref, ea_ref, w_ref, b_ref, o_ref):
    q = jnp.dot(ea_ref[...], w_ref[...],
                preferred_element_type=jnp.float32) + b_ref[...]
    o_ref[...] = jnp.maximum(g_ref[...].astype(jnp.float32) + q,
                             0.0).astype(o_ref.dtype)


def _msg(g, ea, w, b, blk, out_dtype=jnp.float32):
    n = g.shape[0]
    assert n % blk == 0
    return pl.pallas_call(
        _msg_body,
        grid=(n // blk,),
        in_specs=[
            pl.BlockSpec((blk, H), lambda i: (i, 0)),
            pl.BlockSpec((blk, H), lambda i: (i, 0)),
            pl.BlockSpec((H, H), lambda i: (0, 0)),
            pl.BlockSpec((1, H), lambda i: (0, 0)),
        ],
        out_specs=pl.BlockSpec((blk, H), lambda i: (i, 0)),
        out_shape=jax.ShapeDtypeStruct((n, H), out_dtype),
    )(g, ea, w, b.reshape(1, H))


def _mlp_stats_body(x_ref, agg_ref, w1_ref, b1_ref, w2_ref, b2_ref,
                    eps_ref, y_ref, s1_ref, s2_ref):
    h = (1.0 + eps_ref[0]) * x_ref[...] + agg_ref[...]
    t = jnp.maximum(jnp.dot(h, w1_ref[...],
                            preferred_element_type=jnp.float32) + b1_ref[...], 0.0)
    y = jnp.maximum(jnp.dot(t, w2_ref[...],
                            preferred_element_type=jnp.float32) + b2_ref[...], 0.0)
    y_ref[...] = y
    s1_ref[...] = jnp.sum(y, axis=0)[None, None, :]
    s2_ref[...] = jnp.sum(y * y, axis=0)[None, None, :]


def _mlp_stats(x, agg, eps, w1, b1, w2, b2, blk):
    n = x.shape[0]
    assert n % blk == 0
    g = n // blk
    return pl.pallas_call(
        _mlp_stats_body,
        grid=(g,),
        in_specs=[
            pl.BlockSpec((blk, H), lambda i: (i, 0)),
            pl.BlockSpec((blk, H), lambda i: (i, 0)),
            pl.BlockSpec((H, H), lambda i: (0, 0)),
            pl.BlockSpec((1, H), lambda i: (0, 0)),
            pl.BlockSpec((H, H), lambda i: (0, 0)),
            pl.BlockSpec((1, H), lambda i: (0, 0)),
            pl.BlockSpec(memory_space=pltpu.SMEM),
        ],
        out_specs=[
            pl.BlockSpec((blk, H), lambda i: (i, 0)),
            pl.BlockSpec((1, 1, H), lambda i: (i, 0, 0)),
            pl.BlockSpec((1, 1, H), lambda i: (i, 0, 0)),
        ],
        out_shape=[
            jax.ShapeDtypeStruct((n, H), jnp.float32),
            jax.ShapeDtypeStruct((g, 1, H), jnp.float32),
            jax.ShapeDtypeStruct((g, 1, H), jnp.float32),
        ],
    )(x, agg, w1, b1.reshape(1, H), w2, b2.reshape(1, H),
      eps.reshape(1))


def _gfin_body(hn_ref, y_ref, s1_ref, s2_ref, g_ref, b_ref, bp_ref, n_ref,
               z_ref):
    n = n_ref[0]
    mu = jnp.sum(s1_ref[...], axis=0) / n
    var = jnp.sum(s2_ref[...], axis=0) / n - mu * mu
    bn = (y_ref[...] - mu) * jax.lax.rsqrt(var + 1e-5) * g_ref[...] + b_ref[...]
    hn2 = hn_ref[...] + bn
    z_ref[...] = jnp.dot(hn2, bp_ref[...],
                         preferred_element_type=jnp.float32
                         ).astype(jnp.bfloat16)


def _gfin(h_node, y_g, s1, s2, bng, bnb, bp_w, blk):
    n = h_node.shape[0]
    g = n // blk
    nb = s1.shape[0]
    return pl.pallas_call(
        _gfin_body,
        grid=(g,),
        in_specs=[
            pl.BlockSpec((blk, H), lambda i: (i, 0)),
            pl.BlockSpec((blk, H), lambda i: (i, 0)),
            pl.BlockSpec((nb, H), lambda i: (0, 0)),
            pl.BlockSpec((nb, H), lambda i: (0, 0)),
            pl.BlockSpec((1, H), lambda i: (0, 0)),
            pl.BlockSpec((1, H), lambda i: (0, 0)),
            pl.BlockSpec((H, H), lambda i: (0, 0)),
            pl.BlockSpec(memory_space=pltpu.SMEM),
        ],
        out_specs=pl.BlockSpec((blk, H), lambda i: (i, 0)),
        out_shape=jax.ShapeDtypeStruct((n, H), jnp.bfloat16),
    )(h_node, y_g, s1, s2, bng.reshape(1, H), bnb.reshape(1, H), bp_w,
      jnp.full((1,), float(n), jnp.float32))


def _fin_body(hin_ref, y_ref, s1_ref, s2_ref, bng_ref, bnb_ref, hb_ref,
              w1a_ref, w1b_ref, b1_ref, w2_ref, b2_ref, lng_ref, lnb_ref,
              v_ref, n_ref, o_ref):
    n = n_ref[0]
    mu = jnp.sum(s1_ref[...], axis=0) / n
    var = jnp.sum(s2_ref[...], axis=0) / n - mu * mu
    h_local = (y_ref[...] - mu) * jax.lax.rsqrt(var + 1e-5) * bng_ref[...] \
        + bnb_ref[...]
    t = jnp.dot(h_local, w1a_ref[...], preferred_element_type=jnp.float32) \
        + jnp.dot(hb_ref[...].astype(jnp.float32), w1b_ref[...],
                  preferred_element_type=jnp.float32) \
        + b1_ref[...]
    t = 0.5 * t * (1.0 + lax.erf(t * 0.7071067811865476))
    u = jnp.dot(t, w2_ref[...], preferred_element_type=jnp.float32) + b2_ref[...]
    mu_r = jnp.mean(u, axis=-1, keepdims=True)
    var_r = jnp.mean(u * u, axis=-1, keepdims=True) - mu_r * mu_r
    ln = (u - mu_r) * jax.lax.rsqrt(var_r + 1e-5) * lng_ref[...] + lnb_ref[...]
    o_ref[...] = (hin_ref[...] + ln) * v_ref[...]


def _fin(h_in, y_l, s1, s2, bng, bnb, h_bcast, ce_w1, ce_b1, ce_w2, ce_b2,
         ln_g, ln_b, valid_f, blk):
    n = h_in.shape[0]
    g = n // blk
    nb = s1.shape[0]
    return pl.pallas_call(
        _fin_body,
        grid=(g,),
        in_specs=[
            pl.BlockSpec((blk, H), lambda i: (i, 0)),
            pl.BlockSpec((blk, H), lambda i: (i, 0)),
            pl.BlockSpec((nb, H), lambda i: (0, 0)),
            pl.BlockSpec((nb, H), lambda i: (0, 0)),
            pl.BlockSpec((1, H), lambda i: (0, 0)),
            pl.BlockSpec((1, H), lambda i: (0, 0)),
            pl.BlockSpec((blk, H), lambda i: (i, 0)),
            pl.BlockSpec((H, H), lambda i: (0, 0)),
            pl.BlockSpec((H, H), lambda i: (0, 0)),
            pl.BlockSpec((1, H), lambda i: (0, 0)),
            pl.BlockSpec((H, H), lambda i: (0, 0)),
            pl.BlockSpec((1, H), lambda i: (0, 0)),
            pl.BlockSpec((1, H), lambda i: (0, 0)),
            pl.BlockSpec((1, H), lambda i: (0, 0)),
            pl.BlockSpec((blk, 1), lambda i: (i, 0)),
            pl.BlockSpec(memory_space=pltpu.SMEM),
        ],
        out_specs=pl.BlockSpec((blk, H), lambda i: (i, 0)),
        out_shape=jax.ShapeDtypeStruct((n, H), jnp.float32),
    )(h_in, y_l, s1, s2, bng.reshape(1, H), bnb.reshape(1, H), h_bcast,
      ce_w1[:H], ce_w1[H:], ce_b1.reshape(1, H), ce_w2, ce_b2.reshape(1, H),
      ln_g.reshape(1, H), ln_b.reshape(1, H), valid_f,
      jnp.full((1,), float(n), jnp.float32))


def _gather_rows(table, idx):
    return jnp.take(table, idx, axis=0)


def _segsum(vals, idx, n):
    return jax.ops.segment_sum(vals, idx, num_segments=n)


def kernel(h_flat, intra_ei, intra_ea, valid_f, global_ei, global_ea,
           root_flat_idx, node_assign, sub_ids, N_total, S,
           l_ew, l_eb, l_w1, l_b1, l_w2, l_b2, l_eps, l_bng, l_bnb,
           g_ew, g_eb, g_w1, g_b1, g_w2, g_b2, g_eps, g_bng, g_bnb,
           bp_w, ce_w1, ce_b1, ce_w2, ce_b2, ln_g, ln_b):
    M = h_flat.shape[0]
    N = 10000

    h_bf = h_flat.astype(jnp.bfloat16)
    msg = _msg(_gather_rows(h_bf, intra_ei[0]), intra_ea, l_ew, l_eb,
               blk=600, out_dtype=jnp.bfloat16)
    agg = _segsum(msg, intra_ei[1], M).astype(jnp.float32)
    y_l, s1_l, s2_l = _mlp_stats(h_flat, agg, l_eps, l_w1, l_b1, l_w2, l_b2,
                                 blk=400)
    s1_l, s2_l = s1_l.reshape(-1, H), s2_l.reshape(-1, H)

    h_root = _gather_rows(h_flat, root_flat_idx)
    sums = _segsum(h_root, node_assign, N)
    cnt = _segsum(jnp.ones((node_assign.shape[0],), jnp.float32),
                  node_assign, N)
    h_node = sums / jnp.maximum(cnt, 1.0)[:, None]

    gmsg = _msg(_gather_rows(h_node.astype(jnp.bfloat16), global_ei[0]),
                global_ea, g_ew, g_eb, blk=640)
    gagg = _segsum(gmsg, global_ei[1], N)
    y_g, s1_g, s2_g = _mlp_stats(h_node, gagg, g_eps, g_w1, g_b1, g_w2, g_b2,
                                 blk=400)
    s1_g, s2_g = s1_g.reshape(-1, H), s2_g.reshape(-1, H)
    z = _gfin(h_node, y_g, s1_g, s2_g, g_bng, g_bnb, bp_w, blk=400)

    h_bcast = _gather_rows(z, jnp.take(node_assign, sub_ids))

    return _fin(h_flat, y_l, s1_l, s2_l, l_bng, l_bnb, h_bcast,
                ce_w1, ce_b1, ce_w2, ce_b2, ln_g, ln_b, valid_f, blk=400)


# speedup vs baseline: 1.0809x; 1.0809x over previous
import functools
import jax
import jax.numpy as jnp
from jax import lax
from jax.experimental import pallas as pl
from jax.experimental.pallas import tpu as pltpu

H = 128


def _msg_body(g_ref, ea_ref, w_ref, b_ref, o_ref):
    q = jnp.dot(ea_ref[...], w_ref[...],
                preferred_element_type=jnp.float32) + b_ref[...]
    o_ref[...] = jnp.maximum(g_ref[...].astype(jnp.float32) + q,
                             0.0).astype(o_ref.dtype)


def _msg(g, ea, w, b, blk, out_dtype=jnp.float32):
    n = g.shape[0]
    assert n % blk == 0
    return pl.pallas_call(
        _msg_body,
        grid=(n // blk,),
        in_specs=[
            pl.BlockSpec((blk, H), lambda i: (i, 0)),
            pl.BlockSpec((blk, H), lambda i: (i, 0)),
            pl.BlockSpec((H, H), lambda i: (0, 0)),
            pl.BlockSpec((1, H), lambda i: (0, 0)),
        ],
        out_specs=pl.BlockSpec((blk, H), lambda i: (i, 0)),
        out_shape=jax.ShapeDtypeStruct((n, H), out_dtype),
    )(g, ea, w, b.reshape(1, H))


def _mlp_stats_body(x_ref, agg_ref, w1_ref, b1_ref, w2_ref, b2_ref,
                    eps_ref, y_ref, s1_ref, s2_ref):
    h = (1.0 + eps_ref[0]) * x_ref[...] + agg_ref[...]
    t = jnp.maximum(jnp.dot(h, w1_ref[...],
                            preferred_element_type=jnp.float32) + b1_ref[...], 0.0)
    y = jnp.maximum(jnp.dot(t, w2_ref[...],
                            preferred_element_type=jnp.float32) + b2_ref[...], 0.0)
    y_ref[...] = y
    s1_ref[...] = jnp.sum(y, axis=0)[None, None, :]
    s2_ref[...] = jnp.sum(y * y, axis=0)[None, None, :]


def _mlp_stats(x, agg, eps, w1, b1, w2, b2, blk):
    n = x.shape[0]
    assert n % blk == 0
    g = n // blk
    return pl.pallas_call(
        _mlp_stats_body,
        grid=(g,),
        in_specs=[
            pl.BlockSpec((blk, H), lambda i: (i, 0)),
            pl.BlockSpec((blk, H), lambda i: (i, 0)),
            pl.BlockSpec((H, H), lambda i: (0, 0)),
            pl.BlockSpec((1, H), lambda i: (0, 0)),
            pl.BlockSpec((H, H), lambda i: (0, 0)),
            pl.BlockSpec((1, H), lambda i: (0, 0)),
            pl.BlockSpec(memory_space=pltpu.SMEM),
        ],
        out_specs=[
            pl.BlockSpec((blk, H), lambda i: (i, 0)),
            pl.BlockSpec((1, 1, H), lambda i: (i, 0, 0)),
            pl.BlockSpec((1, 1, H), lambda i: (i, 0, 0)),
        ],
        out_shape=[
            jax.ShapeDtypeStruct((n, H), jnp.float32),
            jax.ShapeDtypeStruct((g, 1, H), jnp.float32),
            jax.ShapeDtypeStruct((g, 1, H), jnp.float32),
        ],
    )(x, agg, w1, b1.reshape(1, H), w2, b2.reshape(1, H),
      eps.reshape(1))


def _gfin_body(hn_ref, y_ref, s1_ref, s2_ref, g_ref, b_ref, bp_ref, n_ref,
               z_ref):
    n = n_ref[0]
    mu = jnp.sum(s1_ref[...], axis=0) / n
    var = jnp.sum(s2_ref[...], axis=0) / n - mu * mu
    bn = (y_ref[...] - mu) * jax.lax.rsqrt(var + 1e-5) * g_ref[...] + b_ref[...]
    hn2 = hn_ref[...] + bn
    z_ref[...] = jnp.dot(hn2, bp_ref[...],
                         preferred_element_type=jnp.float32
                         ).astype(jnp.bfloat16)


def _gfin(h_node, y_g, s1, s2, bng, bnb, bp_w, blk):
    n = h_node.shape[0]
    g = n // blk
    nb = s1.shape[0]
    return pl.pallas_call(
        _gfin_body,
        grid=(g,),
        in_specs=[
            pl.BlockSpec((blk, H), lambda i: (i, 0)),
            pl.BlockSpec((blk, H), lambda i: (i, 0)),
            pl.BlockSpec((nb, H), lambda i: (0, 0)),
            pl.BlockSpec((nb, H), lambda i: (0, 0)),
            pl.BlockSpec((1, H), lambda i: (0, 0)),
            pl.BlockSpec((1, H), lambda i: (0, 0)),
            pl.BlockSpec((H, H), lambda i: (0, 0)),
            pl.BlockSpec(memory_space=pltpu.SMEM),
        ],
        out_specs=pl.BlockSpec((blk, H), lambda i: (i, 0)),
        out_shape=jax.ShapeDtypeStruct((n, H), jnp.bfloat16),
    )(h_node, y_g, s1, s2, bng.reshape(1, H), bnb.reshape(1, H), bp_w,
      jnp.full((1,), float(n), jnp.float32))


def _fin_body(hin_ref, y_ref, s1_ref, s2_ref, bng_ref, bnb_ref, hb_ref,
              w1a_ref, w1b_ref, b1_ref, w2_ref, b2_ref, lng_ref, lnb_ref,
              v_ref, n_ref, o_ref):
    n = n_ref[0]
    mu = jnp.sum(s1_ref[...], axis=0) / n
    var = jnp.sum(s2_ref[...], axis=0) / n - mu * mu
    h_local = (y_ref[...] - mu) * jax.lax.rsqrt(var + 1e-5) * bng_ref[...] \
        + bnb_ref[...]
    t = jnp.dot(h_local, w1a_ref[...], preferred_element_type=jnp.float32) \
        + jnp.dot(hb_ref[...].astype(jnp.float32), w1b_ref[...],
                  preferred_element_type=jnp.float32) \
        + b1_ref[...]
    t = 0.5 * t * (1.0 + lax.erf(t * 0.7071067811865476))
    u = jnp.dot(t, w2_ref[...], preferred_element_type=jnp.float32) + b2_ref[...]
    mu_r = jnp.mean(u, axis=-1, keepdims=True)
    var_r = jnp.mean(u * u, axis=-1, keepdims=True) - mu_r * mu_r
    ln = (u - mu_r) * jax.lax.rsqrt(var_r + 1e-5) * lng_ref[...] + lnb_ref[...]
    o_ref[...] = (hin_ref[...] + ln) * v_ref[...]


def _fin(h_in, y_l, s1, s2, bng, bnb, h_bcast, ce_w1, ce_b1, ce_w2, ce_b2,
         ln_g, ln_b, valid_f, blk):
    n = h_in.shape[0]
    g = n // blk
    nb = s1.shape[0]
    return pl.pallas_call(
        _fin_body,
        grid=(g,),
        in_specs=[
            pl.BlockSpec((blk, H), lambda i: (i, 0)),
            pl.BlockSpec((blk, H), lambda i: (i, 0)),
            pl.BlockSpec((nb, H), lambda i: (0, 0)),
            pl.BlockSpec((nb, H), lambda i: (0, 0)),
            pl.BlockSpec((1, H), lambda i: (0, 0)),
            pl.BlockSpec((1, H), lambda i: (0, 0)),
            pl.BlockSpec((blk, H), lambda i: (i, 0)),
            pl.BlockSpec((H, H), lambda i: (0, 0)),
            pl.BlockSpec((H, H), lambda i: (0, 0)),
            pl.BlockSpec((1, H), lambda i: (0, 0)),
            pl.BlockSpec((H, H), lambda i: (0, 0)),
            pl.BlockSpec((1, H), lambda i: (0, 0)),
            pl.BlockSpec((1, H), lambda i: (0, 0)),
            pl.BlockSpec((1, H), lambda i: (0, 0)),
            pl.BlockSpec((blk, 1), lambda i: (i, 0)),
            pl.BlockSpec(memory_space=pltpu.SMEM),
        ],
        out_specs=pl.BlockSpec((blk, H), lambda i: (i, 0)),
        out_shape=jax.ShapeDtypeStruct((n, H), jnp.float32),
    )(h_in, y_l, s1, s2, bng.reshape(1, H), bnb.reshape(1, H), h_bcast,
      ce_w1[:H], ce_w1[H:], ce_b1.reshape(1, H), ce_w2, ce_b2.reshape(1, H),
      ln_g.reshape(1, H), ln_b.reshape(1, H), valid_f,
      jnp.full((1,), float(n), jnp.float32))


def _gather_rows(table, idx):
    return jnp.take(table, idx, axis=0)


def _segsum(vals, idx, n):
    return jax.ops.segment_sum(vals, idx, num_segments=n)


def kernel(h_flat, intra_ei, intra_ea, valid_f, global_ei, global_ea,
           root_flat_idx, node_assign, sub_ids, N_total, S,
           l_ew, l_eb, l_w1, l_b1, l_w2, l_b2, l_eps, l_bng, l_bnb,
           g_ew, g_eb, g_w1, g_b1, g_w2, g_b2, g_eps, g_bng, g_bnb,
           bp_w, ce_w1, ce_b1, ce_w2, ce_b2, ln_g, ln_b):
    M = h_flat.shape[0]
    N = 10000

    h_bf = h_flat.astype(jnp.bfloat16)
    msg = _msg(_gather_rows(h_bf, intra_ei[0]), intra_ea, l_ew, l_eb,
               blk=1200)
    agg = _segsum(msg, intra_ei[1], M)
    y_l, s1_l, s2_l = _mlp_stats(h_flat, agg, l_eps, l_w1, l_b1, l_w2, l_b2,
                                 blk=800)
    s1_l, s2_l = s1_l.reshape(-1, H), s2_l.reshape(-1, H)

    h_root = _gather_rows(h_flat, root_flat_idx)
    sums = _segsum(h_root, node_assign, N)
    cnt = _segsum(jnp.ones((node_assign.shape[0],), jnp.float32),
                  node_assign, N)
    h_node = sums / jnp.maximum(cnt, 1.0)[:, None]

    gmsg = _msg(_gather_rows(h_node.astype(jnp.bfloat16), global_ei[0]),
                global_ea, g_ew, g_eb, blk=640)
    gagg = _segsum(gmsg, global_ei[1], N)
    y_g, s1_g, s2_g = _mlp_stats(h_node, gagg, g_eps, g_w1, g_b1, g_w2, g_b2,
                                 blk=400)
    s1_g, s2_g = s1_g.reshape(-1, H), s2_g.reshape(-1, H)
    z = _gfin(h_node, y_g, s1_g, s2_g, g_bng, g_bnb, bp_w, blk=400)

    h_bcast = _gather_rows(z, jnp.take(node_assign, sub_ids))

    return _fin(h_flat, y_l, s1_l, s2_l, l_bng, l_bnb, h_bcast,
                ce_w1, ce_b1, ce_w2, ce_b2, ln_g, ln_b, valid_f, blk=800)


# speedup vs baseline: 1.1250x; 1.0408x over previous
import functools
import jax
import jax.numpy as jnp
from jax import lax
from jax.experimental import pallas as pl
from jax.experimental.pallas import tpu as pltpu

H = 128


def _msg_body(g_ref, ea_ref, w_ref, b_ref, o_ref):
    q = jnp.dot(ea_ref[...], w_ref[...],
                preferred_element_type=jnp.float32) + b_ref[...]
    o_ref[...] = jnp.maximum(g_ref[...].astype(jnp.float32) + q,
                             0.0).astype(o_ref.dtype)


def _msg(g, ea, w, b, blk, out_dtype=jnp.float32):
    n = g.shape[0]
    assert n % blk == 0
    return pl.pallas_call(
        _msg_body,
        grid=(n // blk,),
        in_specs=[
            pl.BlockSpec((blk, H), lambda i: (i, 0)),
            pl.BlockSpec((blk, H), lambda i: (i, 0)),
            pl.BlockSpec((H, H), lambda i: (0, 0)),
            pl.BlockSpec((1, H), lambda i: (0, 0)),
        ],
        out_specs=pl.BlockSpec((blk, H), lambda i: (i, 0)),
        out_shape=jax.ShapeDtypeStruct((n, H), out_dtype),
    )(g, ea, w, b.reshape(1, H))


def _mlp_stats_body(x_ref, agg_ref, w1_ref, b1_ref, w2_ref, b2_ref,
                    eps_ref, y_ref, s1_ref, s2_ref):
    h = (1.0 + eps_ref[0]) * x_ref[...] + agg_ref[...]
    t = jnp.maximum(jnp.dot(h, w1_ref[...],
                            preferred_element_type=jnp.float32) + b1_ref[...], 0.0)
    y = jnp.maximum(jnp.dot(t, w2_ref[...],
                            preferred_element_type=jnp.float32) + b2_ref[...], 0.0)
    y_ref[...] = y
    s1_ref[...] = jnp.sum(y, axis=0)[None, None, :]
    s2_ref[...] = jnp.sum(y * y, axis=0)[None, None, :]


def _mlp_stats(x, agg, eps, w1, b1, w2, b2, blk):
    n = x.shape[0]
    assert n % blk == 0
    g = n // blk
    return pl.pallas_call(
        _mlp_stats_body,
        grid=(g,),
        in_specs=[
            pl.BlockSpec((blk, H), lambda i: (i, 0)),
            pl.BlockSpec((blk, H), lambda i: (i, 0)),
            pl.BlockSpec((H, H), lambda i: (0, 0)),
            pl.BlockSpec((1, H), lambda i: (0, 0)),
            pl.BlockSpec((H, H), lambda i: (0, 0)),
            pl.BlockSpec((1, H), lambda i: (0, 0)),
            pl.BlockSpec(memory_space=pltpu.SMEM),
        ],
        out_specs=[
            pl.BlockSpec((blk, H), lambda i: (i, 0)),
            pl.BlockSpec((1, 1, H), lambda i: (i, 0, 0)),
            pl.BlockSpec((1, 1, H), lambda i: (i, 0, 0)),
        ],
        out_shape=[
            jax.ShapeDtypeStruct((n, H), jnp.float32),
            jax.ShapeDtypeStruct((g, 1, H), jnp.float32),
            jax.ShapeDtypeStruct((g, 1, H), jnp.float32),
        ],
    )(x, agg, w1, b1.reshape(1, H), w2, b2.reshape(1, H),
      eps.reshape(1))


def _gfin_body(hn_ref, y_ref, s1_ref, s2_ref, g_ref, b_ref, bp_ref, n_ref,
               z_ref):
    n = n_ref[0]
    mu = jnp.sum(s1_ref[...], axis=0) / n
    var = jnp.sum(s2_ref[...], axis=0) / n - mu * mu
    bn = (y_ref[...] - mu) * jax.lax.rsqrt(var + 1e-5) * g_ref[...] + b_ref[...]
    hn2 = hn_ref[...] + bn
    z_ref[...] = jnp.dot(hn2, bp_ref[...],
                         preferred_element_type=jnp.float32
                         ).astype(jnp.bfloat16)


def _gfin(h_node, y_g, s1, s2, bng, bnb, bp_w, blk):
    n = h_node.shape[0]
    g = n // blk
    nb = s1.shape[0]
    return pl.pallas_call(
        _gfin_body,
        grid=(g,),
        in_specs=[
            pl.BlockSpec((blk, H), lambda i: (i, 0)),
            pl.BlockSpec((blk, H), lambda i: (i, 0)),
            pl.BlockSpec((nb, H), lambda i: (0, 0)),
            pl.BlockSpec((nb, H), lambda i: (0, 0)),
            pl.BlockSpec((1, H), lambda i: (0, 0)),
            pl.BlockSpec((1, H), lambda i: (0, 0)),
            pl.BlockSpec((H, H), lambda i: (0, 0)),
            pl.BlockSpec(memory_space=pltpu.SMEM),
        ],
        out_specs=pl.BlockSpec((blk, H), lambda i: (i, 0)),
        out_shape=jax.ShapeDtypeStruct((n, H), jnp.bfloat16),
    )(h_node, y_g, s1, s2, bng.reshape(1, H), bnb.reshape(1, H), bp_w,
      jnp.full((1,), float(n), jnp.float32))


def _fin_body(hin_ref, y_ref, s1_ref, s2_ref, bng_ref, bnb_ref, hb_ref,
              w1a_ref, w1b_ref, b1_ref, w2_ref, b2_ref, lng_ref, lnb_ref,
              v_ref, n_ref, o_ref):
    n = n_ref[0]
    mu = jnp.sum(s1_ref[...], axis=0) / n
    var = jnp.sum(s2_ref[...], axis=0) / n - mu * mu
    h_local = (y_ref[...] - mu) * jax.lax.rsqrt(var + 1e-5) * bng_ref[...] \
        + bnb_ref[...]
    t = jnp.dot(h_local, w1a_ref[...], preferred_element_type=jnp.float32) \
        + jnp.dot(hb_ref[...].astype(jnp.float32), w1b_ref[...],
                  preferred_element_type=jnp.float32) \
        + b1_ref[...]
    t = 0.5 * t * (1.0 + lax.erf(t * 0.7071067811865476))
    u = jnp.dot(t, w2_ref[...], preferred_element_type=jnp.float32) + b2_ref[...]
    mu_r = jnp.mean(u, axis=-1, keepdims=True)
    var_r = jnp.mean(u * u, axis=-1, keepdims=True) - mu_r * mu_r
    ln = (u - mu_r) * jax.lax.rsqrt(var_r + 1e-5) * lng_ref[...] + lnb_ref[...]
    o_ref[...] = (hin_ref[...] + ln) * v_ref[...]


def _fin(h_in, y_l, s1, s2, bng, bnb, h_bcast, ce_w1, ce_b1, ce_w2, ce_b2,
         ln_g, ln_b, valid_f, blk):
    n = h_in.shape[0]
    g = n // blk
    nb = s1.shape[0]
    return pl.pallas_call(
        _fin_body,
        grid=(g,),
        in_specs=[
            pl.BlockSpec((blk, H), lambda i: (i, 0)),
            pl.BlockSpec((blk, H), lambda i: (i, 0)),
            pl.BlockSpec((nb, H), lambda i: (0, 0)),
            pl.BlockSpec((nb, H), lambda i: (0, 0)),
            pl.BlockSpec((1, H), lambda i: (0, 0)),
            pl.BlockSpec((1, H), lambda i: (0, 0)),
            pl.BlockSpec((blk, H), lambda i: (i, 0)),
            pl.BlockSpec((H, H), lambda i: (0, 0)),
            pl.BlockSpec((H, H), lambda i: (0, 0)),
            pl.BlockSpec((1, H), lambda i: (0, 0)),
            pl.BlockSpec((H, H), lambda i: (0, 0)),
            pl.BlockSpec((1, H), lambda i: (0, 0)),
            pl.BlockSpec((1, H), lambda i: (0, 0)),
            pl.BlockSpec((1, H), lambda i: (0, 0)),
            pl.BlockSpec((blk, 1), lambda i: (i, 0)),
            pl.BlockSpec(memory_space=pltpu.SMEM),
        ],
        out_specs=pl.BlockSpec((blk, H), lambda i: (i, 0)),
        out_shape=jax.ShapeDtypeStruct((n, H), jnp.float32),
    )(h_in, y_l, s1, s2, bng.reshape(1, H), bnb.reshape(1, H), h_bcast,
      ce_w1[:H], ce_w1[H:], ce_b1.reshape(1, H), ce_w2, ce_b2.reshape(1, H),
      ln_g.reshape(1, H), ln_b.reshape(1, H), valid_f,
      jnp.full((1,), float(n), jnp.float32))


def _gather_rows(table, idx):
    return jnp.take(table, idx, axis=0)


def _segsum(vals, idx, n):
    return jax.ops.segment_sum(vals, idx, num_segments=n)


def kernel(h_flat, intra_ei, intra_ea, valid_f, global_ei, global_ea,
           root_flat_idx, node_assign, sub_ids, N_total, S,
           l_ew, l_eb, l_w1, l_b1, l_w2, l_b2, l_eps, l_bng, l_bnb,
           g_ew, g_eb, g_w1, g_b1, g_w2, g_b2, g_eps, g_bng, g_bnb,
           bp_w, ce_w1, ce_b1, ce_w2, ce_b2, ln_g, ln_b):
    M = h_flat.shape[0]
    N = 10000

    h_bf = h_flat.astype(jnp.bfloat16)
    msg = _msg(_gather_rows(h_bf, intra_ei[0]), intra_ea, l_ew, l_eb,
               blk=3000)
    agg = _segsum(msg, intra_ei[1], M)
    y_l, s1_l, s2_l = _mlp_stats(h_flat, agg, l_eps, l_w1, l_b1, l_w2, l_b2,
                                 blk=2000)
    s1_l, s2_l = s1_l.reshape(-1, H), s2_l.reshape(-1, H)

    h_root = _gather_rows(h_flat, root_flat_idx)
    sums = _segsum(h_root, node_assign, N)
    cnt = _segsum(jnp.ones((node_assign.shape[0],), jnp.float32),
                  node_assign, N)
    h_node = sums / jnp.maximum(cnt, 1.0)[:, None]

    gmsg = _msg(_gather_rows(h_node.astype(jnp.bfloat16), global_ei[0]),
                global_ea, g_ew, g_eb, blk=640)
    gagg = _segsum(gmsg, global_ei[1], N)
    y_g, s1_g, s2_g = _mlp_stats(h_node, gagg, g_eps, g_w1, g_b1, g_w2, g_b2,
                                 blk=400)
    s1_g, s2_g = s1_g.reshape(-1, H), s2_g.reshape(-1, H)
    z = _gfin(h_node, y_g, s1_g, s2_g, g_bng, g_bnb, bp_w, blk=400)

    h_bcast = _gather_rows(z, jnp.take(node_assign, sub_ids))

    return _fin(h_flat, y_l, s1_l, s2_l, l_bng, l_bnb, h_bcast,
                ce_w1, ce_b1, ce_w2, ce_b2, ln_g, ln_b, valid_f, blk=2000)


# speedup vs baseline: 1.1619x; 1.0328x over previous
import functools
import jax
import jax.numpy as jnp
from jax import lax
from jax.experimental import pallas as pl
from jax.experimental.pallas import tpu as pltpu

H = 128


def _msg_body(g_ref, ea_ref, w_ref, b_ref, o_ref):
    q = jnp.dot(ea_ref[...], w_ref[...],
                preferred_element_type=jnp.float32) + b_ref[...]
    o_ref[...] = jnp.maximum(g_ref[...].astype(jnp.float32) + q,
                             0.0).astype(o_ref.dtype)


def _msg(g, ea, w, b, blk, out_dtype=jnp.float32):
    n = g.shape[0]
    assert n % blk == 0
    return pl.pallas_call(
        _msg_body,
        grid=(n // blk,),
        in_specs=[
            pl.BlockSpec((blk, H), lambda i: (i, 0)),
            pl.BlockSpec((blk, H), lambda i: (i, 0)),
            pl.BlockSpec((H, H), lambda i: (0, 0)),
            pl.BlockSpec((1, H), lambda i: (0, 0)),
        ],
        out_specs=pl.BlockSpec((blk, H), lambda i: (i, 0)),
        out_shape=jax.ShapeDtypeStruct((n, H), out_dtype),
    )(g, ea, w, b.reshape(1, H))


def _mlp_stats_body(x_ref, agg_ref, w1_ref, b1_ref, w2_ref, b2_ref,
                    eps_ref, y_ref, s1_ref, s2_ref):
    h = (1.0 + eps_ref[0]) * x_ref[...] + agg_ref[...]
    t = jnp.maximum(jnp.dot(h, w1_ref[...],
                            preferred_element_type=jnp.float32) + b1_ref[...], 0.0)
    y = jnp.maximum(jnp.dot(t, w2_ref[...],
                            preferred_element_type=jnp.float32) + b2_ref[...], 0.0)
    y_ref[...] = y
    s1_ref[...] = jnp.sum(y, axis=0)[None, None, :]
    s2_ref[...] = jnp.sum(y * y, axis=0)[None, None, :]


def _mlp_stats(x, agg, eps, w1, b1, w2, b2, blk):
    n = x.shape[0]
    assert n % blk == 0
    g = n // blk
    return pl.pallas_call(
        _mlp_stats_body,
        grid=(g,),
        in_specs=[
            pl.BlockSpec((blk, H), lambda i: (i, 0)),
            pl.BlockSpec((blk, H), lambda i: (i, 0)),
            pl.BlockSpec((H, H), lambda i: (0, 0)),
            pl.BlockSpec((1, H), lambda i: (0, 0)),
            pl.BlockSpec((H, H), lambda i: (0, 0)),
            pl.BlockSpec((1, H), lambda i: (0, 0)),
            pl.BlockSpec(memory_space=pltpu.SMEM),
        ],
        out_specs=[
            pl.BlockSpec((blk, H), lambda i: (i, 0)),
            pl.BlockSpec((1, 1, H), lambda i: (i, 0, 0)),
            pl.BlockSpec((1, 1, H), lambda i: (i, 0, 0)),
        ],
        out_shape=[
            jax.ShapeDtypeStruct((n, H), jnp.float32),
            jax.ShapeDtypeStruct((g, 1, H), jnp.float32),
            jax.ShapeDtypeStruct((g, 1, H), jnp.float32),
        ],
    )(x, agg, w1, b1.reshape(1, H), w2, b2.reshape(1, H),
      eps.reshape(1))


def _gfin_body(hn_ref, y_ref, s1_ref, s2_ref, g_ref, b_ref, bp_ref, n_ref,
               z_ref):
    n = n_ref[0]
    mu = jnp.sum(s1_ref[...], axis=0) / n
    var = jnp.sum(s2_ref[...], axis=0) / n - mu * mu
    bn = (y_ref[...] - mu) * jax.lax.rsqrt(var + 1e-5) * g_ref[...] + b_ref[...]
    hn2 = hn_ref[...] + bn
    z_ref[...] = jnp.dot(hn2, bp_ref[...],
                         preferred_element_type=jnp.float32
                         ).astype(jnp.bfloat16)


def _gfin(h_node, y_g, s1, s2, bng, bnb, bp_w, blk):
    n = h_node.shape[0]
    g = n // blk
    nb = s1.shape[0]
    return pl.pallas_call(
        _gfin_body,
        grid=(g,),
        in_specs=[
            pl.BlockSpec((blk, H), lambda i: (i, 0)),
            pl.BlockSpec((blk, H), lambda i: (i, 0)),
            pl.BlockSpec((nb, H), lambda i: (0, 0)),
            pl.BlockSpec((nb, H), lambda i: (0, 0)),
            pl.BlockSpec((1, H), lambda i: (0, 0)),
            pl.BlockSpec((1, H), lambda i: (0, 0)),
            pl.BlockSpec((H, H), lambda i: (0, 0)),
            pl.BlockSpec(memory_space=pltpu.SMEM),
        ],
        out_specs=pl.BlockSpec((blk, H), lambda i: (i, 0)),
        out_shape=jax.ShapeDtypeStruct((n, H), jnp.bfloat16),
    )(h_node, y_g, s1, s2, bng.reshape(1, H), bnb.reshape(1, H), bp_w,
      jnp.full((1,), float(n), jnp.float32))


def _fin_body(hin_ref, y_ref, s1_ref, s2_ref, bng_ref, bnb_ref, hb_ref,
              w1a_ref, w1b_ref, b1_ref, w2_ref, b2_ref, lng_ref, lnb_ref,
              v_ref, n_ref, o_ref):
    n = n_ref[0]
    mu = jnp.sum(s1_ref[...], axis=0) / n
    var = jnp.sum(s2_ref[...], axis=0) / n - mu * mu
    h_local = (y_ref[...] - mu) * jax.lax.rsqrt(var + 1e-5) * bng_ref[...] \
        + bnb_ref[...]
    t = jnp.dot(h_local, w1a_ref[...], preferred_element_type=jnp.float32) \
        + jnp.dot(hb_ref[...].astype(jnp.float32), w1b_ref[...],
                  preferred_element_type=jnp.float32) \
        + b1_ref[...]
    t = 0.5 * t * (1.0 + lax.erf(t * 0.7071067811865476))
    u = jnp.dot(t, w2_ref[...], preferred_element_type=jnp.float32) + b2_ref[...]
    mu_r = jnp.mean(u, axis=-1, keepdims=True)
    var_r = jnp.mean(u * u, axis=-1, keepdims=True) - mu_r * mu_r
    ln = (u - mu_r) * jax.lax.rsqrt(var_r + 1e-5) * lng_ref[...] + lnb_ref[...]
    o_ref[...] = (hin_ref[...] + ln) * v_ref[...]


def _fin(h_in, y_l, s1, s2, bng, bnb, h_bcast, ce_w1, ce_b1, ce_w2, ce_b2,
         ln_g, ln_b, valid_f, blk):
    n = h_in.shape[0]
    g = n // blk
    nb = s1.shape[0]
    return pl.pallas_call(
        _fin_body,
        grid=(g,),
        in_specs=[
            pl.BlockSpec((blk, H), lambda i: (i, 0)),
            pl.BlockSpec((blk, H), lambda i: (i, 0)),
            pl.BlockSpec((nb, H), lambda i: (0, 0)),
            pl.BlockSpec((nb, H), lambda i: (0, 0)),
            pl.BlockSpec((1, H), lambda i: (0, 0)),
            pl.BlockSpec((1, H), lambda i: (0, 0)),
            pl.BlockSpec((blk, H), lambda i: (i, 0)),
            pl.BlockSpec((H, H), lambda i: (0, 0)),
            pl.BlockSpec((H, H), lambda i: (0, 0)),
            pl.BlockSpec((1, H), lambda i: (0, 0)),
            pl.BlockSpec((H, H), lambda i: (0, 0)),
            pl.BlockSpec((1, H), lambda i: (0, 0)),
            pl.BlockSpec((1, H), lambda i: (0, 0)),
            pl.BlockSpec((1, H), lambda i: (0, 0)),
            pl.BlockSpec((blk, 1), lambda i: (i, 0)),
            pl.BlockSpec(memory_space=pltpu.SMEM),
        ],
        out_specs=pl.BlockSpec((blk, H), lambda i: (i, 0)),
        out_shape=jax.ShapeDtypeStruct((n, H), jnp.float32),
    )(h_in, y_l, s1, s2, bng.reshape(1, H), bnb.reshape(1, H), h_bcast,
      ce_w1[:H], ce_w1[H:], ce_b1.reshape(1, H), ce_w2, ce_b2.reshape(1, H),
      ln_g.reshape(1, H), ln_b.reshape(1, H), valid_f,
      jnp.full((1,), float(n), jnp.float32))


def _gather_rows(table, idx):
    return jnp.take(table, idx, axis=0)


def _segsum(vals, idx, n):
    return jax.ops.segment_sum(vals, idx, num_segments=n)


def kernel(h_flat, intra_ei, intra_ea, valid_f, global_ei, global_ea,
           root_flat_idx, node_assign, sub_ids, N_total, S,
           l_ew, l_eb, l_w1, l_b1, l_w2, l_b2, l_eps, l_bng, l_bnb,
           g_ew, g_eb, g_w1, g_b1, g_w2, g_b2, g_eps, g_bng, g_bnb,
           bp_w, ce_w1, ce_b1, ce_w2, ce_b2, ln_g, ln_b):
    M = h_flat.shape[0]
    N = 10000

    h_bf = h_flat.astype(jnp.bfloat16)
    msg = _msg(_gather_rows(h_bf, intra_ei[0]), intra_ea, l_ew, l_eb,
               blk=6000)
    agg = _segsum(msg, intra_ei[1], M)
    y_l, s1_l, s2_l = _mlp_stats(h_flat, agg, l_eps, l_w1, l_b1, l_w2, l_b2,
                                 blk=4000)
    s1_l, s2_l = s1_l.reshape(-1, H), s2_l.reshape(-1, H)

    h_root = _gather_rows(h_flat, root_flat_idx)
    sums = _segsum(h_root, node_assign, N)
    cnt = _segsum(jnp.ones((node_assign.shape[0],), jnp.float32),
                  node_assign, N)
    h_node = sums / jnp.maximum(cnt, 1.0)[:, None]

    gmsg = _msg(_gather_rows(h_node.astype(jnp.bfloat16), global_ei[0]),
                global_ea, g_ew, g_eb, blk=3200)
    gagg = _segsum(gmsg, global_ei[1], N)
    y_g, s1_g, s2_g = _mlp_stats(h_node, gagg, g_eps, g_w1, g_b1, g_w2, g_b2,
                                 blk=400)
    s1_g, s2_g = s1_g.reshape(-1, H), s2_g.reshape(-1, H)
    z = _gfin(h_node, y_g, s1_g, s2_g, g_bng, g_bnb, bp_w, blk=400)

    h_bcast = _gather_rows(z, jnp.take(node_assign, sub_ids))

    return _fin(h_flat, y_l, s1_l, s2_l, l_bng, l_bnb, h_bcast,
                ce_w1, ce_b1, ce_w2, ce_b2, ln_g, ln_b, valid_f, blk=4000)
